# Initial kernel scaffold; baseline (speedup 1.0000x reference)
import jax
import jax.numpy as jnp
from jax.experimental import pallas as pl


def kernel(relative_attention_bias, seq_length):
    raise NotImplementedError("write your pallas kernel here")


# speedup vs baseline: 232.8139x; 232.8139x over previous
import math

import jax
import jax.numpy as jnp
from jax.experimental import pallas as pl
from jax.experimental.pallas import tpu as pltpu

_NUM_BUCKETS = 32
_MAX_DISTANCE = 512
_SEQ = 2048
_H = 12
_DP = 4352
_DT = 4096
_BI = 128
_G = 8
_M = 16


def _bucket_of(rel):
    nb2 = _NUM_BUCKETS // 2
    me = _NUM_BUCKETS // 4
    rb = (rel > 0).astype(jnp.int32) * nb2
    rb = rb + (rel < 0).astype(jnp.int32) * nb2
    rp = jnp.abs(rel)
    is_small = rp < me
    rp_safe = jnp.maximum(rp, 1).astype(jnp.float32)
    rp_if_large = me + (
        jnp.log(rp_safe / me) / math.log(_MAX_DISTANCE / me) * (nb2 - me)
    ).astype(jnp.int32)
    rp_if_large = jnp.minimum(rp_if_large, nb2 - 1)
    return rb + jnp.where(is_small, rp, rp_if_large)


def _body(table_t_ref, out_ref, t_ref):
    pid = pl.program_id(0)

    @pl.when(pid == 0)
    def _build_t():
        u = jax.lax.broadcasted_iota(jnp.int32, (1, _DP), 1)
        bucket = _bucket_of(u - (_SEQ - 1))
        b_iota = jax.lax.broadcasted_iota(jnp.int32, (_NUM_BUCKETS, _DP), 0)
        onehot = (b_iota == bucket).astype(jnp.float32)
        dvpad = jax.lax.dot_general(
            table_t_ref[...], onehot, (((1,), (0,)), ((), ())),
            preferred_element_type=jnp.float32)
        x3 = jnp.broadcast_to(dvpad[:, None, :], (_H, _G, _DP))
        for m in range(_M):
            rolled = pltpu.roll(x3, -(8 * m + 7) % _DP, axis=2,
                                stride=1, stride_axis=1)
            t_ref[m] = rolled[:, :, :_DT]

    q = (_SEQ // _BI - 1) - pid
    start = q * _BI
    for g in range(_BI // _G):
        out_ref[:, g * _G:(g + 1) * _G, :] = \
            t_ref[(_M - 1) - g, :, :, pl.ds(start, _SEQ)]


def kernel(relative_attention_bias, seq_length):
    del seq_length
    table_t = relative_attention_bias.T
    grid = (_SEQ // _BI,)
    out = pl.pallas_call(
        _body,
        grid=grid,
        in_specs=[pl.BlockSpec((_H, _NUM_BUCKETS), lambda i: (0, 0))],
        out_specs=pl.BlockSpec((_H, _BI, _SEQ), lambda i: (0, i, 0)),
        out_shape=jax.ShapeDtypeStruct((_H, _SEQ, _SEQ), jnp.float32),
        scratch_shapes=[pltpu.VMEM((_M, _H, _G, _DT), jnp.float32)],
        compiler_params=pltpu.CompilerParams(
            dimension_semantics=("arbitrary",),
        ),
    )(table_t)
    return out
